# baseline (speedup 1.0000x reference)
import functools

import jax
import jax.numpy as jnp
from jax import lax
from jax.experimental import pallas as pl
from jax.experimental.pallas import tpu as pltpu
from jax.experimental.pallas import tpu_sc as plsc

N = 10000
N_PAD = 10240
E = 320000
D = 128
C = 128
NC = 2
NS = 16
NW = NC * NS
NJ = 80
NCHUNK = NW * NJ
E_PAD = NCHUNK * C
ROWS_PER_TILE = N_PAD // NS
DEGW = 16


def _fill2d(ref, rows, width, value):
    vec = jnp.full((16,), value, dtype=jnp.float32)

    def body(r, _):
        for k in range(width // 16):
            ref[r, pl.ds(16 * k, 16)] = vec
        return 0

    lax.fori_loop(0, rows, body, 0)


def _make_sc_agg(with_deg: bool):
    G = 4 if with_deg else 8
    NG = NJ // G

    out_type = [jax.ShapeDtypeStruct((NC, N_PAD, D), jnp.float32)]
    scratch = [
        pltpu.VMEM((G, 2, C), jnp.int32),
        pltpu.VMEM((G, 2, C), jnp.int32),
        pltpu.VMEM((C, D), jnp.float32),
        pltpu.VMEM((C, D), jnp.float32),
        pltpu.VMEM_SHARED((N_PAD, D), jnp.float32),
        pltpu.SemaphoreType.DMA,
        pltpu.SemaphoreType.DMA,
        pltpu.SemaphoreType.DMA,
    ]
    if with_deg:
        out_type.append(jax.ShapeDtypeStruct((NC, N_PAD, DEGW), jnp.float32))
        scratch.append(pltpu.VMEM((C, DEGW), jnp.float32))
        scratch.append(pltpu.VMEM_SHARED((N_PAD, DEGW), jnp.float32))

    mesh = plsc.VectorSubcoreMesh(core_axis_name="c", subcore_axis_name="s")

    def body(x_hbm, idx_hbm, *refs):
        if with_deg:
            (agg_hbm, deg_hbm, idx0, idx1, rows0, rows1, acc_sh,
             sem0, sem1, semi, ones_v, deg_sh) = refs
        else:
            (agg_hbm, idx0, idx1, rows0, rows1, acc_sh,
             sem0, sem1, semi) = refs

        cid = lax.axis_index("c")
        sid = lax.axis_index("s")
        wid = cid * NS + sid
        base = sid * ROWS_PER_TILE
        rows = (rows0, rows1)
        sems = (sem0, sem1)

        def load_idx(grp, ibuf):
            pltpu.async_copy(idx_hbm.at[pl.ds(wid * NJ + grp * G, G)],
                             ibuf, semi)

        def idx_wait(ibuf):
            pltpu.make_async_copy(idx_hbm.at[pl.ds(0, G)], ibuf, semi).wait()

        load_idx(0, idx0)
        _fill2d(rows0, C, D, 0.0)
        for q in range(ROWS_PER_TILE // C):
            pltpu.sync_copy(rows0, acc_sh.at[pl.ds(base + q * C, C)])
        if with_deg:
            _fill2d(ones_v, C, DEGW, 0.0)
            for q in range(ROWS_PER_TILE // C):
                pltpu.sync_copy(ones_v, deg_sh.at[pl.ds(base + q * C, C)])
            _fill2d(ones_v, C, DEGW, 1.0)
        idx_wait(idx0)
        plsc.subcore_barrier()

        def gather(ibuf, k, rbuf, sem):
            pltpu.async_copy(x_hbm.at[ibuf.at[k, 0]], rbuf, sem)

        def gwait(rbuf, sem):
            pltpu.make_async_copy(x_hbm.at[idx0.at[0, 0]], rbuf, sem).wait()

        def scatter(ibuf, k, rbuf):
            pltpu.sync_copy(rbuf, acc_sh.at[ibuf.at[k, 1]], add=True)
            if with_deg:
                pltpu.sync_copy(ones_v, deg_sh.at[ibuf.at[k, 1]], add=True)

        def run_group(grp, cur, nxt, last):
            @pl.when(jnp.logical_not(last))
            def _():
                load_idx(grp + 1, nxt)

            gather(cur, 0, rows[0], sems[0])
            for k in range(G):
                if k + 1 < G:
                    gather(cur, k + 1, rows[(k + 1) % 2], sems[(k + 1) % 2])
                gwait(rows[k % 2], sems[k % 2])
                scatter(cur, k, rows[k % 2])

            @pl.when(jnp.logical_not(last))
            def _():
                idx_wait(nxt)

        def pair_body(t, _):
            run_group(2 * t, idx0, idx1, jnp.bool_(False))
            run_group(2 * t + 1, idx1, idx0, t >= NG // 2 - 1)
            return 0

        lax.fori_loop(0, NG // 2, pair_body, 0)
        plsc.subcore_barrier()

        pltpu.sync_copy(acc_sh.at[pl.ds(base, ROWS_PER_TILE)],
                        agg_hbm.at[cid, pl.ds(base, ROWS_PER_TILE)])
        if with_deg:
            pltpu.sync_copy(deg_sh.at[pl.ds(base, ROWS_PER_TILE)],
                            deg_hbm.at[cid, pl.ds(base, ROWS_PER_TILE)])

    out = tuple(out_type) if with_deg else out_type[0]
    return pl.kernel(
        body, out_type=out, mesh=mesh, scratch_types=scratch,
        compiler_params=pltpu.CompilerParams(use_tc_tiling_on_sc=False))


_sc_agg_deg = _make_sc_agg(True)
_sc_agg = _make_sc_agg(False)

_BLK = 1000


def _dense_body(relu, x_ref, aggp_ref, degp_ref, ws_ref, wn_ref, b_ref, o_ref):
    agg = aggp_ref[0] + aggp_ref[1]
    deg = jnp.sum(degp_ref[0] + degp_ref[1], axis=1, keepdims=True) / DEGW
    aggn = agg / jnp.maximum(deg, 1.0)
    o = (jnp.dot(x_ref[...], ws_ref[...], preferred_element_type=jnp.float32)
         + jnp.dot(aggn, wn_ref[...], preferred_element_type=jnp.float32)
         + b_ref[...])
    o_ref[...] = jnp.maximum(o, 0.0) if relu else o


def _dense(x, aggp, degp, Ws, Wn, b, relu):
    return pl.pallas_call(
        functools.partial(_dense_body, relu),
        grid=(N // _BLK,),
        in_specs=[
            pl.BlockSpec((_BLK, D), lambda i: (i, 0)),
            pl.BlockSpec((NC, _BLK, D), lambda i: (0, i, 0)),
            pl.BlockSpec((NC, _BLK, DEGW), lambda i: (0, i, 0)),
            pl.BlockSpec((D, D), lambda i: (0, 0)),
            pl.BlockSpec((D, D), lambda i: (0, 0)),
            pl.BlockSpec((1, D), lambda i: (0, 0)),
        ],
        out_specs=pl.BlockSpec((_BLK, D), lambda i: (i, 0)),
        out_shape=jax.ShapeDtypeStruct((N, D), jnp.float32),
    )(x, aggp, degp, Ws, Wn, b.reshape(1, D))


def kernel(inputs, edge_index, W_self1, W_neigh1, b1, W_self2, W_neigh2, b2,
           W_self3, W_neigh3, b3):
    pad = E_PAD - E
    srcp = jnp.concatenate([edge_index[0], jnp.zeros((pad,), jnp.int32)])
    dstp = jnp.concatenate([edge_index[1], jnp.full((pad,), N, jnp.int32)])
    idx2 = jnp.stack([srcp.reshape(NCHUNK, C), dstp.reshape(NCHUNK, C)],
                     axis=1)

    aggp1, degp = _sc_agg_deg(inputs, idx2)
    h1 = _dense(inputs, aggp1, degp, W_self1, W_neigh1, b1, relu=True)
    aggp2 = _sc_agg(h1, idx2)
    h2 = _dense(h1, aggp2, degp, W_self2, W_neigh2, b2, relu=True)
    aggp3 = _sc_agg(h2, idx2)
    return _dense(h2, aggp3, degp, W_self3, W_neigh3, b3, relu=False)
